# baseline (speedup 1.0000x reference)
import jax
import jax.numpy as jnp
from jax.experimental import pallas as pl
from jax.experimental.pallas import tpu as pltpu

_N = 16777216
_R, _C = _N // 128, 128
_G = 32
_CR = _R // _G
_RESCH = 25
_ROT = 4
_SLOTS = _RESCH + _ROT
_REF = _G - _SLOTS
_SUB = 64


def _slot(j):
    if isinstance(j, int):
        return j if j < _RESCH else _RESCH + (j % _ROT)
    return jnp.where(j < _RESCH, j, _RESCH + (j % _ROT))


def _fetch(x_hbm, buf, sems, j):
    return pltpu.make_async_copy(
        x_hbm.at[pl.ds(j * _CR, _CR), :],
        buf.at[pl.ds(_slot(j) * _CR, _CR), :],
        sems.at[_slot(j)],
    )


def _put(o_hbm, buf, sems, j):
    return pltpu.make_async_copy(
        buf.at[pl.ds(_slot(j) * _CR, _CR), :],
        o_hbm.at[pl.ds(j * _CR, _CR), :],
        sems.at[_slot(j)],
    )


def _body(denom_ref, x_hbm, o_hbm, buf, in_sems, out_sems):
    for j in range(_SLOTS):
        _fetch(x_hbm, buf, in_sems, j).start()

    def _chunk_red(i, carry):
        _fetch(x_hbm, buf, in_sems, i).wait()
        base = _slot(i) * _CR

        def _red(k, c2):
            a, b = c2
            v = buf[pl.ds(base + k * _SUB, _SUB), :]
            return jnp.minimum(a, v), jnp.maximum(b, v)

        carry = jax.lax.fori_loop(0, _CR // _SUB, _red, carry)

        @pl.when(jnp.logical_and(i + _ROT >= _SLOTS, i + _ROT < _G))
        def _():
            _fetch(x_hbm, buf, in_sems, i + _ROT).start()

        return carry

    inf = jnp.float32(jnp.inf)
    cmn, cmx = jax.lax.fori_loop(
        0,
        _G,
        _chunk_red,
        (jnp.full((_SUB, _C), inf), jnp.full((_SUB, _C), -inf)),
    )
    mn = jnp.min(cmn)
    sc = (jnp.max(cmx) - mn) / denom_ref[0]
    inv = 1.0 / sc

    def _chunk_q(j, carry):
        cj = jnp.where(
            j < _RESCH, j, jnp.where(j < _RESCH + _ROT, j + _REF, j - _ROT)
        )
        sj = jnp.where(
            j < _RESCH,
            j,
            jnp.where(
                j < _RESCH + _ROT,
                _RESCH + ((j + _REF) % _ROT),
                j - (_RESCH + _ROT),
            ),
        )

        @pl.when(j >= _RESCH + _ROT)
        def _():
            pltpu.make_async_copy(
                x_hbm.at[pl.ds(cj * _CR, _CR), :],
                buf.at[pl.ds(sj * _CR, _CR), :],
                in_sems.at[sj],
            ).wait()

        base = sj * _CR

        def _quant(k, c2):
            r = pl.ds(base + k * _SUB, _SUB)
            buf[r, :] = jnp.round((buf[r, :] - mn) * inv) * sc + mn
            return c2

        jax.lax.fori_loop(0, _CR // _SUB, _quant, 0)
        pltpu.make_async_copy(
            buf.at[pl.ds(sj * _CR, _CR), :],
            o_hbm.at[pl.ds(cj * _CR, _CR), :],
            out_sems.at[sj],
        ).start()

        @pl.when(jnp.logical_and(j >= _REF, j < 2 * _REF))
        def _():
            s = j - _REF
            pltpu.make_async_copy(
                buf.at[pl.ds(s * _CR, _CR), :],
                o_hbm.at[pl.ds(s * _CR, _CR), :],
                out_sems.at[s],
            ).wait()
            pltpu.make_async_copy(
                x_hbm.at[pl.ds((_RESCH + s) * _CR, _CR), :],
                buf.at[pl.ds(s * _CR, _CR), :],
                in_sems.at[s],
            ).start()

        return carry

    jax.lax.fori_loop(0, _G, _chunk_q, 0)

    for s in range(_SLOTS):
        pltpu.make_async_copy(
            buf.at[pl.ds(s * _CR, _CR), :],
            o_hbm.at[pl.ds(s * _CR, _CR), :],
            out_sems.at[s],
        ).wait()


def kernel(tensor, bits):
    x = tensor.reshape(_R, _C)
    denom = jnp.asarray((2 ** bits) - 1, dtype=jnp.float32).reshape(1)

    y = pl.pallas_call(
        _body,
        in_specs=[
            pl.BlockSpec(memory_space=pltpu.SMEM),
            pl.BlockSpec(memory_space=pl.ANY),
        ],
        out_specs=pl.BlockSpec(memory_space=pl.ANY),
        out_shape=jax.ShapeDtypeStruct((_R, _C), jnp.float32),
        scratch_shapes=[
            pltpu.VMEM((_SLOTS * _CR, _C), jnp.float32),
            pltpu.SemaphoreType.DMA((_SLOTS,)),
            pltpu.SemaphoreType.DMA((_SLOTS,)),
        ],
    )(denom, x)

    return y.reshape(tensor.shape)
